# baseline (speedup 1.0000x reference)
import functools

import jax
import jax.numpy as jnp
from jax import lax
from jax.experimental import pallas as pl
from jax.experimental.pallas import tpu as pltpu
from jax.experimental.pallas import tpu_sc as plsc

LANES = 16
WIDE = 128


def _make_sc_transpose(vocab, emb, n_workers):
    n_full, tail = divmod(vocab, WIDE)
    vocab_pad = (n_full + (1 if tail else 0)) * WIDE
    bands_per_w = (n_full + n_workers - 1) // n_workers
    mesh = plsc.VectorSubcoreMesh(core_axis_name="c", subcore_axis_name="s")
    n_cores = 2

    @functools.partial(
        pl.kernel,
        mesh=mesh,
        out_type=jax.ShapeDtypeStruct((vocab_pad, WIDE), jnp.float32),
        scratch_types=[
            pltpu.VMEM((emb, WIDE + 1), jnp.float32),
            pltpu.VMEM((emb, WIDE + 1), jnp.float32),
            pltpu.VMEM((WIDE, WIDE), jnp.float32),
            pltpu.VMEM((WIDE, WIDE), jnp.float32),
            pltpu.VMEM((tail if tail else 8, emb), jnp.float32),
            pltpu.SemaphoreType.DMA,
            pltpu.SemaphoreType.DMA,
            pltpu.SemaphoreType.DMA,
            pltpu.SemaphoreType.DMA,
        ],
        compiler_params=pltpu.CompilerParams(needs_layout_passes=False),
    )
    def sc_tp(tt_hbm, tail_hbm, wide_hbm, in0, in1, out0, out1, tail_buf, si0, si1, so0, so1):
        wid = lax.axis_index("s") * n_cores + lax.axis_index("c")
        ins = (in0, in1)
        outs = (out0, out1)
        sis = (si0, si1)
        sos = (so0, so1)
        iota = lax.iota(jnp.int32, LANES)

        if tail:
            @pl.when(wid == 0)
            def _():
                pltpu.sync_copy(tail_hbm, tail_buf)

                def cp(v, _):
                    for j in range(emb // LANES):
                        out0[v, pl.ds(j * LANES, LANES)] = tail_buf[
                            v, pl.ds(j * LANES, LANES)
                        ]
                    return 0

                lax.fori_loop(0, tail, cp, 0)
                pltpu.sync_copy(
                    out0, wide_hbm.at[pl.ds(n_full * WIDE, WIDE), :]
                )

        def band_id(i):
            return wid + n_workers * i

        def fetch(i, p):
            pltpu.async_copy(
                tt_hbm.at[:, pl.ds(band_id(i) * WIDE, WIDE)],
                ins[p].at[:, pl.ds(0, WIDE)],
                sis[p],
            )

        def wait_fetch(p):
            pltpu.make_async_copy(
                tt_hbm.at[:, pl.ds(0, WIDE)],
                ins[p].at[:, pl.ds(0, WIDE)],
                sis[p],
            ).wait()

        VU = 4

        def transpose(p):
            def body(i, _):
                for u in range(VU):
                    v = i * VU + u
                    vidx = jnp.full((LANES,), v, jnp.int32)
                    for j in range(emb // LANES):
                        vals = plsc.load_gather(
                            ins[p], [iota + j * LANES, vidx]
                        )
                        outs[p][v, pl.ds(j * LANES, LANES)] = vals
                return 0

            lax.fori_loop(0, WIDE // VU, body, 0)

        def put(i, p):
            pltpu.async_copy(
                outs[p], wide_hbm.at[pl.ds(band_id(i) * WIDE, WIDE), :], sos[p]
            )

        def wait_put(p):
            pltpu.make_async_copy(
                outs[p], wide_hbm.at[pl.ds(0, WIDE), :], sos[p]
            ).wait()

        fetch(0, 0)

        def step(i2, _):
            for p in range(2):
                i = i2 * 2 + p

                @pl.when((i >= 2) & (band_id(i - 2) < n_full))
                def _():
                    wait_put(p)

                @pl.when(band_id(i) < n_full)
                def _():
                    @pl.when(band_id(i + 1) < n_full)
                    def _():
                        fetch(i + 1, 1 - p)

                    wait_fetch(p)
                    transpose(p)
                    put(i, p)
            return 0

        n_steps = (bands_per_w + 1) // 2
        lax.fori_loop(0, n_steps, step, 0)
        for last in (2 * n_steps - 2, 2 * n_steps - 1):
            if last >= 0:
                @pl.when(band_id(last) < n_full)
                def _():
                    wait_put(last % 2)

    return sc_tp


def _make_sc_pool(batch, seq, emb, n_workers):
    bpw = batch // n_workers
    mesh = plsc.VectorSubcoreMesh(core_axis_name="c", subcore_axis_name="s")
    n_cores = 2
    nj = emb // LANES

    @functools.partial(
        pl.kernel,
        mesh=mesh,
        out_type=jax.ShapeDtypeStruct((batch, emb), jnp.float32),
        scratch_types=[
            pltpu.VMEM((bpw, seq), jnp.int32),
            pltpu.VMEM((seq, WIDE), jnp.float32),
            pltpu.VMEM((seq, WIDE), jnp.float32),
            pltpu.VMEM((bpw, emb), jnp.float32),
            pltpu.SemaphoreType.DMA,
            pltpu.SemaphoreType.DMA,
        ],
    )
    def sc_pool(idx_hbm, wide_hbm, out_hbm, idx_v, rows0, rows1, out_v, sem0, sem1):
        wid = lax.axis_index("s") * n_cores + lax.axis_index("c")
        base = wid * bpw
        pltpu.sync_copy(idx_hbm.at[pl.ds(base, bpw)], idx_v)

        rows = (rows0, rows1)
        sems = (sem0, sem1)
        chunks = []
        off = 0
        while off < seq:
            c = min(128, seq - off)
            chunks.append((off, c))
            off += c

        def fire(bb, rbuf, sem):
            for off, c in chunks:
                pltpu.async_copy(
                    wide_hbm.at[idx_v.at[bb, pl.ds(off, c)]],
                    rbuf.at[pl.ds(off, c)],
                    sem,
                )

        def wait_fetch(bb, rbuf, sem):
            for off, c in chunks:
                pltpu.make_async_copy(
                    wide_hbm.at[idx_v.at[bb, pl.ds(off, c)]],
                    rbuf.at[pl.ds(off, c)],
                    sem,
                ).wait()

        def reduce_rows(rbuf, b):
            def inner(i, accs):
                new = list(accs)
                for k in range(8):
                    s8 = i * 8 + k
                    for j in range(nj):
                        new[j] = new[j] + rbuf[s8, pl.ds(j * LANES, LANES)]
                return tuple(new)

            accs = tuple(jnp.zeros((LANES,), jnp.float32) for _ in range(nj))
            accs = lax.fori_loop(0, seq // 8, inner, accs)
            for j in range(nj):
                out_v[b, pl.ds(j * LANES, LANES)] = accs[j]

        fire(0, rows0, sem0)

        def outer(b2, _):
            b = b2 * 2
            for p in range(2):
                bb = b + p

                @pl.when(bb + 1 < bpw)
                def _():
                    fire(bb + 1, rows[1 - p], sems[1 - p])

                wait_fetch(bb, rows[p], sems[p])
                reduce_rows(rows[p], bb)
            return 0

        lax.fori_loop(0, bpw // 2, outer, 0)
        pltpu.sync_copy(out_v, out_hbm.at[pl.ds(base, bpw)])

    return sc_pool


def _dense_body(inv_seq, x_ref, w1_ref, b1_ref, w2_ref, b2_ref, o_ref):
    x = x_ref[...] * inv_seq
    h = jnp.dot(x, w1_ref[...], preferred_element_type=jnp.float32) + b1_ref[...]
    h = jnp.maximum(h, 0.0)
    logits = jnp.dot(h, w2_ref[...], preferred_element_type=jnp.float32) + b2_ref[...]
    m = jnp.max(logits, axis=-1, keepdims=True)
    e = jnp.exp(logits - m)
    o_ref[...] = e / jnp.sum(e, axis=-1, keepdims=True)


def kernel(inputs, emb_table, W1, b1, W2, b2):
    batch, seq = inputs.shape
    vocab, emb = emb_table.shape
    n_classes = W2.shape[1]

    idx = inputs.astype(jnp.int32)
    tail = vocab % WIDE
    tail_src = emb_table[vocab - (tail if tail else 8):]
    wide = _make_sc_transpose(vocab, emb, 32)(emb_table.T, tail_src)
    pooled = _make_sc_pool(batch, seq, emb, 32)(idx, wide)

    dense = pl.pallas_call(
        functools.partial(_dense_body, 1.0 / seq),
        out_shape=jax.ShapeDtypeStruct((batch, n_classes), jnp.float32),
    )
    return dense(pooled, W1, b1.reshape(1, -1), W2, b2.reshape(1, -1))


# speedup vs baseline: 3.3677x; 3.3677x over previous
import functools

import jax
import jax.numpy as jnp
from jax import lax
from jax.experimental import pallas as pl
from jax.experimental.pallas import tpu as pltpu
from jax.experimental.pallas import tpu_sc as plsc

LANES = 16
WIDE = 128


def _make_sc_transpose(vocab, emb, n_workers):
    n_full, tail = divmod(vocab, WIDE)
    vocab_pad = (n_full + (1 if tail else 0)) * WIDE
    bands_per_w = (n_full + n_workers - 1) // n_workers
    mesh = plsc.VectorSubcoreMesh(core_axis_name="c", subcore_axis_name="s")
    n_cores = 2

    @functools.partial(
        pl.kernel,
        mesh=mesh,
        out_type=jax.ShapeDtypeStruct((vocab_pad, WIDE), jnp.float32),
        scratch_types=[
            pltpu.VMEM((emb, WIDE + 1), jnp.float32),
            pltpu.VMEM((emb, WIDE + 1), jnp.float32),
            pltpu.VMEM((WIDE, WIDE), jnp.float32),
            pltpu.VMEM((WIDE, WIDE), jnp.float32),
            pltpu.VMEM((tail if tail else 8, emb), jnp.float32),
            pltpu.SemaphoreType.DMA,
            pltpu.SemaphoreType.DMA,
            pltpu.SemaphoreType.DMA,
            pltpu.SemaphoreType.DMA,
        ],
        compiler_params=pltpu.CompilerParams(needs_layout_passes=False),
    )
    def sc_tp(tt_hbm, tail_hbm, wide_hbm, in0, in1, out0, out1, tail_buf, si0, si1, so0, so1):
        wid = lax.axis_index("s") * n_cores + lax.axis_index("c")
        ins = (in0, in1)
        outs = (out0, out1)
        sis = (si0, si1)
        sos = (so0, so1)
        iota = lax.iota(jnp.int32, LANES)

        if tail:
            @pl.when(wid == 0)
            def _():
                pltpu.sync_copy(tail_hbm, tail_buf)

                def cp(v, _):
                    for j in range(emb // LANES):
                        out0[v, pl.ds(j * LANES, LANES)] = tail_buf[
                            v, pl.ds(j * LANES, LANES)
                        ]
                    return 0

                lax.fori_loop(0, tail, cp, 0)
                pltpu.sync_copy(
                    out0, wide_hbm.at[pl.ds(n_full * WIDE, WIDE), :]
                )

        def band_id(i):
            return wid + n_workers * i

        def fetch(i, p):
            pltpu.async_copy(
                tt_hbm.at[:, pl.ds(band_id(i) * WIDE, WIDE)],
                ins[p].at[:, pl.ds(0, WIDE)],
                sis[p],
            )

        def wait_fetch(p):
            pltpu.make_async_copy(
                tt_hbm.at[:, pl.ds(0, WIDE)],
                ins[p].at[:, pl.ds(0, WIDE)],
                sis[p],
            ).wait()

        def transpose(p):
            @functools.partial(plsc.parallel_loop, 0, WIDE, unroll=4)
            def _(v):
                vidx = jnp.full((LANES,), v, jnp.int32)
                for j in range(emb // LANES):
                    vals = plsc.load_gather(ins[p], [iota + j * LANES, vidx])
                    outs[p][v, pl.ds(j * LANES, LANES)] = vals

        def put(i, p):
            pltpu.async_copy(
                outs[p], wide_hbm.at[pl.ds(band_id(i) * WIDE, WIDE), :], sos[p]
            )

        def wait_put(p):
            pltpu.make_async_copy(
                outs[p], wide_hbm.at[pl.ds(0, WIDE), :], sos[p]
            ).wait()

        fetch(0, 0)

        def step(i2, _):
            for p in range(2):
                i = i2 * 2 + p

                @pl.when((i >= 2) & (band_id(i - 2) < n_full))
                def _():
                    wait_put(p)

                @pl.when(band_id(i) < n_full)
                def _():
                    @pl.when(band_id(i + 1) < n_full)
                    def _():
                        fetch(i + 1, 1 - p)

                    wait_fetch(p)
                    transpose(p)
                    put(i, p)
            return 0

        n_steps = (bands_per_w + 1) // 2
        lax.fori_loop(0, n_steps, step, 0)
        for last in (2 * n_steps - 2, 2 * n_steps - 1):
            if last >= 0:
                @pl.when(band_id(last) < n_full)
                def _():
                    wait_put(last % 2)

    return sc_tp


def _make_sc_pool(batch, seq, emb, n_workers):
    bpw = batch // n_workers
    mesh = plsc.VectorSubcoreMesh(core_axis_name="c", subcore_axis_name="s")
    n_cores = 2
    nj = emb // LANES

    @functools.partial(
        pl.kernel,
        mesh=mesh,
        out_type=jax.ShapeDtypeStruct((batch, emb), jnp.float32),
        scratch_types=[
            pltpu.VMEM((bpw, seq), jnp.int32),
            pltpu.VMEM((seq, WIDE), jnp.float32),
            pltpu.VMEM((seq, WIDE), jnp.float32),
            pltpu.VMEM((bpw, emb), jnp.float32),
            pltpu.SemaphoreType.DMA,
            pltpu.SemaphoreType.DMA,
        ],
    )
    def sc_pool(idx_hbm, wide_hbm, out_hbm, idx_v, rows0, rows1, out_v, sem0, sem1):
        wid = lax.axis_index("s") * n_cores + lax.axis_index("c")
        base = wid * bpw
        pltpu.sync_copy(idx_hbm.at[pl.ds(base, bpw)], idx_v)

        rows = (rows0, rows1)
        sems = (sem0, sem1)
        chunks = []
        off = 0
        while off < seq:
            c = min(128, seq - off)
            chunks.append((off, c))
            off += c

        def fire(bb, rbuf, sem):
            for off, c in chunks:
                pltpu.async_copy(
                    wide_hbm.at[idx_v.at[bb, pl.ds(off, c)]],
                    rbuf.at[pl.ds(off, c)],
                    sem,
                )

        def wait_fetch(bb, rbuf, sem):
            for off, c in chunks:
                pltpu.make_async_copy(
                    wide_hbm.at[idx_v.at[bb, pl.ds(off, c)]],
                    rbuf.at[pl.ds(off, c)],
                    sem,
                ).wait()

        def reduce_rows(rbuf, b):
            def inner(i, accs):
                new = list(accs)
                for k in range(8):
                    s8 = i * 8 + k
                    for j in range(nj):
                        new[j] = new[j] + rbuf[s8, pl.ds(j * LANES, LANES)]
                return tuple(new)

            accs = tuple(jnp.zeros((LANES,), jnp.float32) for _ in range(nj))
            accs = lax.fori_loop(0, seq // 8, inner, accs)
            for j in range(nj):
                out_v[b, pl.ds(j * LANES, LANES)] = accs[j]

        fire(0, rows0, sem0)

        def outer(b2, _):
            b = b2 * 2
            for p in range(2):
                bb = b + p

                @pl.when(bb + 1 < bpw)
                def _():
                    fire(bb + 1, rows[1 - p], sems[1 - p])

                wait_fetch(bb, rows[p], sems[p])
                reduce_rows(rows[p], bb)
            return 0

        lax.fori_loop(0, bpw // 2, outer, 0)
        pltpu.sync_copy(out_v, out_hbm.at[pl.ds(base, bpw)])

    return sc_pool


def _dense_body(inv_seq, x_ref, w1_ref, b1_ref, w2_ref, b2_ref, o_ref):
    x = x_ref[...] * inv_seq
    h = jnp.dot(x, w1_ref[...], preferred_element_type=jnp.float32) + b1_ref[...]
    h = jnp.maximum(h, 0.0)
    logits = jnp.dot(h, w2_ref[...], preferred_element_type=jnp.float32) + b2_ref[...]
    m = jnp.max(logits, axis=-1, keepdims=True)
    e = jnp.exp(logits - m)
    o_ref[...] = e / jnp.sum(e, axis=-1, keepdims=True)


def kernel(inputs, emb_table, W1, b1, W2, b2):
    batch, seq = inputs.shape
    vocab, emb = emb_table.shape
    n_classes = W2.shape[1]

    idx = inputs.astype(jnp.int32)
    tail = vocab % WIDE
    tail_src = emb_table[vocab - (tail if tail else 8):]
    wide = _make_sc_transpose(vocab, emb, 32)(emb_table.T, tail_src)
    pooled = _make_sc_pool(batch, seq, emb, 32)(idx, wide)

    dense = pl.pallas_call(
        functools.partial(_dense_body, 1.0 / seq),
        out_shape=jax.ShapeDtypeStruct((batch, n_classes), jnp.float32),
    )
    return dense(pooled, W1, b1.reshape(1, -1), W2, b2.reshape(1, -1))
